# baseline (speedup 1.0000x reference)
import jax
import jax.numpy as jnp
from jax import lax
from jax.experimental import pallas as pl
from jax.experimental.pallas import tpu as pltpu

_K = 50
_INT_MIN = -2147483648
_INT_MAX = 2147483647


def _favg(lo, hi):
    return (lo >> 1) + (hi >> 1) + (lo & hi & 1)


def _sortable(x):
    b = lax.bitcast_convert_type(x, jnp.int32)
    return b ^ ((b >> 31) & jnp.int32(0x7FFFFFFF))


def _filter_body(temp_ref, topk_ref, topp_ref, logits_ref, gum_ref,
                 out_ref, tok_ref, s_ref, e_ref):
    R, C = out_ref.shape
    neg_inf = jnp.float32(-jnp.inf)
    temp = temp_ref[0]
    topk = topk_ref[0]
    topp = topp_ref[0]

    v = logits_ref[...] / jnp.maximum(temp, jnp.float32(1e-8))
    out_ref[...] = v
    s_ref[...] = _sortable(v)

    D = 4
    BINS = 1024
    NCH = C // BINS
    vals, cnts = [], []
    bound = None
    for d in range(D):
        s3 = s_ref[...].reshape(R, NCH, BINS)
        masked = s3 if d == 0 else jnp.where(
            s3 < bound[:, None, :], s3, _INT_MIN)
        lmax = jnp.max(masked, axis=1)
        cnt = jnp.sum((s3 == lmax[:, None, :]).astype(jnp.int32), axis=1)
        vals.append(lmax)
        cnts.append(cnt)
        bound = lmax
    row_max_s = jnp.max(vals[0], axis=1, keepdims=True)
    maxbound = jnp.max(vals[D - 1], axis=1, keepdims=True)

    def bisc(_, carry):
        lo, hi = carry
        mid = _favg(lo, hi)
        tot = jnp.zeros((R, 1), jnp.int32)
        for d in range(D):
            tot = tot + jnp.sum(jnp.where(vals[d] > mid, cnts[d], 0),
                                axis=1, keepdims=True)
        pred = tot < _K
        return (jnp.where(pred, lo, mid + 1), jnp.where(pred, mid, hi))

    lo0 = jnp.full((R, 1), _INT_MIN, jnp.int32)
    Tc, _ = lax.fori_loop(0, 32, bisc, (lo0, row_max_s))
    conv = maxbound <= Tc

    def bis_cond(carry):
        lo, hi = carry
        return jnp.any(lo < hi)

    def bis1(carry):
        lo, hi = carry
        mid = _favg(lo, hi)
        cnt = jnp.sum((s_ref[...] > mid).astype(jnp.int32),
                      axis=1, keepdims=True)
        pred = cnt < _K
        return (jnp.where(pred, lo, mid + 1), jnp.where(pred, mid, hi))

    T, _ = lax.while_loop(bis_cond, bis1,
                          (Tc, jnp.where(conv, Tc, row_max_s)))

    surv = jnp.logical_or(topk <= 0, s_ref[...] >= T)
    m_bits = row_max_s ^ ((row_max_s >> 31) & jnp.int32(0x7FFFFFFF))
    m = lax.bitcast_convert_type(m_bits, jnp.float32)
    e = jnp.where(surv, jnp.exp(out_ref[...] - m), jnp.float32(0.0))
    e_ref[...] = e
    Z = jnp.sum(e, axis=1, keepdims=True)
    PZ = topp * Z

    evs = []
    for d in range(D):
        vb = vals[d] ^ ((vals[d] >> 31) & jnp.int32(0x7FFFFFFF))
        vf = lax.bitcast_convert_type(vb, jnp.float32)
        evs.append(jnp.where(cnts[d] > 0,
                             cnts[d].astype(jnp.float32) * jnp.exp(vf - m),
                             jnp.float32(0.0)))

    def bisc2(_, carry):
        lo, hi = carry
        mid = _favg(lo, hi)
        F = jnp.zeros((R, 1), jnp.float32)
        for d in range(D):
            F = F + jnp.sum(jnp.where(vals[d] > mid, evs[d], 0.0),
                            axis=1, keepdims=True)
        pred = F < PZ
        return (jnp.where(pred, lo, mid + 1), jnp.where(pred, mid, hi))

    s_cc, _ = lax.fori_loop(0, 32, bisc2, (Tc, row_max_s))

    def bis2(carry):
        lo, hi = carry
        mid = _favg(lo, hi)
        F = jnp.sum(jnp.where(s_ref[...] > mid, e_ref[...], 0.0),
                    axis=1, keepdims=True)
        pred = F < PZ
        return (jnp.where(pred, lo, mid + 1), jnp.where(pred, mid, hi))

    s_c, _ = lax.while_loop(bis_cond, bis2,
                            (jnp.where(conv, s_cc, T),
                             jnp.where(conv, s_cc, row_max_s)))

    A = jnp.sum(jnp.where(s_ref[...] > s_c, e_ref[...], 0.0),
                axis=1, keepdims=True)
    e_c = jnp.max(jnp.where(s_ref[...] == s_c, e_ref[...], 0.0),
                  axis=1, keepdims=True)
    q = (PZ - A) / e_c

    iota = lax.broadcasted_iota(jnp.int32, (R, C), 1)
    grp0 = s_ref[...] == s_c
    g_cnt = jnp.sum(grp0.astype(jnp.int32), axis=1, keepdims=True)
    min_idx = jnp.min(jnp.where(grp0, iota, _INT_MAX), axis=1, keepdims=True)
    max_idx = jnp.max(jnp.where(grp0, iota, -1), axis=1, keepdims=True)
    g_f = g_cnt.astype(jnp.float32)
    slow = (q > jnp.float32(1.0)) & (q <= g_f - jnp.float32(1.0))
    need_slow = jnp.any(slow)

    def bis3_cond(carry):
        lo, hi = carry
        return need_slow & jnp.any(lo < hi)

    def bis3(carry):
        lo, hi = carry
        mid = _favg(lo, hi)
        it = lax.broadcasted_iota(jnp.int32, (R, C), 1)
        gr = s_ref[...] == s_c
        cnt = jnp.sum((gr & (it <= mid)).astype(jnp.int32),
                      axis=1, keepdims=True)
        pred = cnt.astype(jnp.float32) >= q
        return (jnp.where(pred, lo, mid + 1), jnp.where(pred, mid, hi))

    ilo0 = jnp.zeros((R, 1), jnp.int32)
    ihi0 = jnp.full((R, 1), jnp.int32(C - 1), jnp.int32)
    slow_idx, _ = lax.while_loop(bis3_cond, bis3, (ilo0, ihi0))
    fast_idx = jnp.where(q <= jnp.float32(1.0), min_idx, max_idx)
    cut_idx = jnp.where(slow, slow_idx, fast_idx)

    sv = s_ref[...]
    iota = lax.broadcasted_iota(jnp.int32, (R, C), 1)
    keep_p = (sv > s_c) | ((sv == s_c) & (iota <= cut_idx))
    keep = surv & jnp.logical_or(topp >= jnp.float32(1.0), keep_p)
    outv = jnp.where(keep, out_ref[...], neg_inf)
    y = outv + gum_ref[...]
    ymax = jnp.max(y, axis=1, keepdims=True)
    tok = jnp.min(jnp.where(y == ymax, iota, _INT_MAX),
                  axis=1, keepdims=True)
    out_ref[...] = outv
    tok_ref[...] = tok


def kernel(logits, temperature, top_k, top_p):
    B, V = logits.shape
    Vp = ((V + 1023) // 1024) * 1024
    pad = Vp - V
    RB = 8 if B % 8 == 0 else B

    gum = jax.random.gumbel(jax.random.key(1), (B, V), logits.dtype)
    lp = jnp.pad(logits, ((0, 0), (0, pad)),
                 constant_values=-jnp.inf) if pad else logits
    gp = jnp.pad(gum, ((0, 0), (0, pad))) if pad else gum

    temp = jnp.asarray(temperature, jnp.float32).reshape(1)
    topk = jnp.asarray(top_k, jnp.int32).reshape(1)
    topp = jnp.asarray(top_p, jnp.float32).reshape(1)

    filtered, tokens = pl.pallas_call(
        _filter_body,
        grid=(B // RB,),
        in_specs=[
            pl.BlockSpec(memory_space=pltpu.SMEM),
            pl.BlockSpec(memory_space=pltpu.SMEM),
            pl.BlockSpec(memory_space=pltpu.SMEM),
            pl.BlockSpec((RB, Vp), lambda i: (i, 0)),
            pl.BlockSpec((RB, Vp), lambda i: (i, 0)),
        ],
        out_specs=[
            pl.BlockSpec((RB, Vp), lambda i: (i, 0)),
            pl.BlockSpec((RB, 1), lambda i: (i, 0)),
        ],
        out_shape=[
            jax.ShapeDtypeStruct((B, Vp), jnp.float32),
            jax.ShapeDtypeStruct((B, 1), jnp.int32),
        ],
        scratch_shapes=[
            pltpu.VMEM((RB, Vp), jnp.int32),
            pltpu.VMEM((RB, Vp), jnp.float32),
        ],
    )(temp, topk, topp, lp, gp)

    return filtered[:, :V], tokens.reshape(B)


# speedup vs baseline: 1.3842x; 1.3842x over previous
import jax
import jax.numpy as jnp
from jax import lax
from jax.experimental import pallas as pl
from jax.experimental.pallas import tpu as pltpu

_K = 50
_INT_MIN = -2147483648
_INT_MAX = 2147483647


def _favg(lo, hi):
    return (lo >> 1) + (hi >> 1) + (lo & hi & 1)


def _sortable(x):
    b = lax.bitcast_convert_type(x, jnp.int32)
    return b ^ ((b >> 31) & jnp.int32(0x7FFFFFFF))


def _filter_body(temp_ref, topk_ref, topp_ref, logits_ref, gum_ref,
                 out_ref, tok_ref, s_ref, e_ref):
    R, C = out_ref.shape
    neg_inf = jnp.float32(-jnp.inf)
    temp = temp_ref[0]
    topk = topk_ref[0]
    topp = topp_ref[0]

    v = logits_ref[...] / jnp.maximum(temp, jnp.float32(1e-8))
    out_ref[...] = v
    s_ref[...] = _sortable(v)

    D = 6
    NCH = C // 128
    vals, cnts = [], []
    bound = None
    for d in range(D):
        s3 = s_ref[...].reshape(R, NCH, 128)
        masked = s3 if d == 0 else jnp.where(
            s3 < bound[:, None, :], s3, _INT_MIN)
        lmax = jnp.max(masked, axis=1)
        cnt = jnp.sum((s3 == lmax[:, None, :]).astype(jnp.int32), axis=1)
        vals.append(lmax)
        cnts.append(cnt)
        bound = lmax
    row_max_s = jnp.max(vals[0], axis=1, keepdims=True)
    maxbound = jnp.max(vals[D - 1], axis=1, keepdims=True)

    def bisc(_, carry):
        lo, hi = carry
        mid = _favg(lo, hi)
        tot = jnp.zeros((R, 1), jnp.int32)
        for d in range(D):
            tot = tot + jnp.sum(jnp.where(vals[d] > mid, cnts[d], 0),
                                axis=1, keepdims=True)
        pred = tot < _K
        return (jnp.where(pred, lo, mid + 1), jnp.where(pred, mid, hi))

    lo0 = jnp.full((R, 1), _INT_MIN, jnp.int32)
    Tc, _ = lax.fori_loop(0, 32, bisc, (lo0, row_max_s))
    conv = maxbound <= Tc

    def bis_cond(carry):
        lo, hi = carry
        return jnp.any(lo < hi)

    def bis1(carry):
        lo, hi = carry
        mid = _favg(lo, hi)
        cnt = jnp.sum((s_ref[...] > mid).astype(jnp.int32),
                      axis=1, keepdims=True)
        pred = cnt < _K
        return (jnp.where(pred, lo, mid + 1), jnp.where(pred, mid, hi))

    T, _ = lax.while_loop(bis_cond, bis1,
                          (Tc, jnp.where(conv, Tc, row_max_s)))

    surv = jnp.logical_or(topk <= 0, s_ref[...] >= T)
    m_bits = row_max_s ^ ((row_max_s >> 31) & jnp.int32(0x7FFFFFFF))
    m = lax.bitcast_convert_type(m_bits, jnp.float32)
    e = jnp.where(surv, jnp.exp(out_ref[...] - m), jnp.float32(0.0))
    e_ref[...] = e
    Z = jnp.sum(e, axis=1, keepdims=True)
    PZ = topp * Z

    evs = []
    for d in range(D):
        vb = vals[d] ^ ((vals[d] >> 31) & jnp.int32(0x7FFFFFFF))
        vf = lax.bitcast_convert_type(vb, jnp.float32)
        evs.append(jnp.where(cnts[d] > 0,
                             cnts[d].astype(jnp.float32) * jnp.exp(vf - m),
                             jnp.float32(0.0)))

    def bisc2(_, carry):
        lo, hi = carry
        mid = _favg(lo, hi)
        F = jnp.zeros((R, 1), jnp.float32)
        for d in range(D):
            F = F + jnp.sum(jnp.where(vals[d] > mid, evs[d], 0.0),
                            axis=1, keepdims=True)
        pred = F < PZ
        return (jnp.where(pred, lo, mid + 1), jnp.where(pred, mid, hi))

    s_cc, _ = lax.fori_loop(0, 32, bisc2, (Tc, row_max_s))

    def bis2(carry):
        lo, hi = carry
        mid = _favg(lo, hi)
        F = jnp.sum(jnp.where(s_ref[...] > mid, e_ref[...], 0.0),
                    axis=1, keepdims=True)
        pred = F < PZ
        return (jnp.where(pred, lo, mid + 1), jnp.where(pred, mid, hi))

    s_c, _ = lax.while_loop(bis_cond, bis2,
                            (jnp.where(conv, s_cc, T),
                             jnp.where(conv, s_cc, row_max_s)))

    A = jnp.sum(jnp.where(s_ref[...] > s_c, e_ref[...], 0.0),
                axis=1, keepdims=True)
    e_c = jnp.max(jnp.where(s_ref[...] == s_c, e_ref[...], 0.0),
                  axis=1, keepdims=True)
    q = (PZ - A) / e_c

    iota = lax.broadcasted_iota(jnp.int32, (R, C), 1)
    grp0 = s_ref[...] == s_c
    g_cnt = jnp.sum(grp0.astype(jnp.int32), axis=1, keepdims=True)
    min_idx = jnp.min(jnp.where(grp0, iota, _INT_MAX), axis=1, keepdims=True)
    max_idx = jnp.max(jnp.where(grp0, iota, -1), axis=1, keepdims=True)
    g_f = g_cnt.astype(jnp.float32)
    slow = (q > jnp.float32(1.0)) & (q <= g_f - jnp.float32(1.0))
    need_slow = jnp.any(slow)

    def bis3_cond(carry):
        lo, hi = carry
        return need_slow & jnp.any(lo < hi)

    def bis3(carry):
        lo, hi = carry
        mid = _favg(lo, hi)
        it = lax.broadcasted_iota(jnp.int32, (R, C), 1)
        gr = s_ref[...] == s_c
        cnt = jnp.sum((gr & (it <= mid)).astype(jnp.int32),
                      axis=1, keepdims=True)
        pred = cnt.astype(jnp.float32) >= q
        return (jnp.where(pred, lo, mid + 1), jnp.where(pred, mid, hi))

    ilo0 = jnp.zeros((R, 1), jnp.int32)
    ihi0 = jnp.full((R, 1), jnp.int32(C - 1), jnp.int32)
    slow_idx, _ = lax.while_loop(bis3_cond, bis3, (ilo0, ihi0))
    fast_idx = jnp.where(q <= jnp.float32(1.0), min_idx, max_idx)
    cut_idx = jnp.where(slow, slow_idx, fast_idx)

    sv = s_ref[...]
    iota = lax.broadcasted_iota(jnp.int32, (R, C), 1)
    keep_p = (sv > s_c) | ((sv == s_c) & (iota <= cut_idx))
    keep = surv & jnp.logical_or(topp >= jnp.float32(1.0), keep_p)
    outv = jnp.where(keep, out_ref[...], neg_inf)
    y = outv + gum_ref[...]
    ymax = jnp.max(y, axis=1, keepdims=True)
    tok = jnp.min(jnp.where(y == ymax, iota, _INT_MAX),
                  axis=1, keepdims=True)
    out_ref[...] = outv
    tok_ref[...] = tok


def kernel(logits, temperature, top_k, top_p):
    B, V = logits.shape
    Vp = ((V + 127) // 128) * 128
    pad = Vp - V
    RB = 8 if B % 8 == 0 else B

    gum = jax.random.gumbel(jax.random.key(1), (B, V), logits.dtype)
    lp = jnp.pad(logits, ((0, 0), (0, pad)),
                 constant_values=-jnp.inf) if pad else logits
    gp = jnp.pad(gum, ((0, 0), (0, pad))) if pad else gum

    temp = jnp.asarray(temperature, jnp.float32).reshape(1)
    topk = jnp.asarray(top_k, jnp.int32).reshape(1)
    topp = jnp.asarray(top_p, jnp.float32).reshape(1)

    filtered, tokens = pl.pallas_call(
        _filter_body,
        grid=(B // RB,),
        in_specs=[
            pl.BlockSpec(memory_space=pltpu.SMEM),
            pl.BlockSpec(memory_space=pltpu.SMEM),
            pl.BlockSpec(memory_space=pltpu.SMEM),
            pl.BlockSpec((RB, Vp), lambda i: (i, 0)),
            pl.BlockSpec((RB, Vp), lambda i: (i, 0)),
        ],
        out_specs=[
            pl.BlockSpec((RB, Vp), lambda i: (i, 0)),
            pl.BlockSpec((RB, 1), lambda i: (i, 0)),
        ],
        out_shape=[
            jax.ShapeDtypeStruct((B, Vp), jnp.float32),
            jax.ShapeDtypeStruct((B, 1), jnp.int32),
        ],
        scratch_shapes=[
            pltpu.VMEM((RB, Vp), jnp.int32),
            pltpu.VMEM((RB, Vp), jnp.float32),
        ],
    )(temp, topk, topp, lp, gp)

    return filtered[:, :V], tokens.reshape(B)
